# baseline (speedup 1.0000x reference)
import functools

import jax
import jax.numpy as jnp
from jax import lax
from jax.experimental import pallas as pl
from jax.experimental.pallas import tpu as pltpu
from jax.experimental.pallas import tpu_sc as plsc

VX, VY = 0.4, 0.4
XMIN, YMIN = -51.2, -51.2
GX, GY = 256, 256
C = 64
NBATCH = 2
N = 100000
NSEG = GX * GY

NS = 16
NGRP = 16
GRP = 6272
NPB = NGRP * GRP
NPTOT = NBATCH * NPB
BLK = 2048
NBLK = NPTOT // BLK
BLKS_PER_BATCH = NPB // BLK
SB = 512

_SC_PARAMS = pltpu.CompilerParams(
    use_tc_tiling_on_sc=False, needs_layout_passes=False)


@functools.cache
def _mesh():
    return plsc.VectorSubcoreMesh(
        core_axis_name="c", subcore_axis_name="s",
        num_cores=2, num_subcores=NS)


def _prep_body(p0, s0_ref, a0_ref, s1_ref, a1_ref):
    i = pl.program_id(0)
    inb = (i % BLKS_PER_BATCH) * BLK
    lane = lax.broadcasted_iota(jnp.int32, (1, BLK), 1)
    valid = (inb + lane) < N

    def one(p, s_ref, a_ref):
        x = p[0:1, :]
        y = p[1:2, :]
        z = p[2:3, :]
        cx = jnp.clip(jnp.floor((x - XMIN) / VX).astype(jnp.int32), 0, GX - 1)
        cy = jnp.clip(jnp.floor((y - YMIN) / VY).astype(jnp.int32), 0, GY - 1)
        s_ref[0] = jnp.where(valid, cy * GX + cx, 0)
        cxc = (cx.astype(jnp.float32) + 0.5) * VX + XMIN
        cyc = (cy.astype(jnp.float32) + 0.5) * VY + YMIN
        ones = jnp.ones((1, BLK), jnp.float32)
        zpad = jnp.zeros((2, BLK), jnp.float32)
        a = jnp.concatenate([ones, x, y, z, cxc, cyc, zpad], axis=0)
        a_ref[...] = jnp.where(valid, a, 0.0)

    one(p0[0], s0_ref, a0_ref)
    one(p0[1], s1_ref, a1_ref)


def _prep(P):
    pspec = pl.BlockSpec((2, 8, BLK), lambda i: (0, 0, i))
    aspec = pl.BlockSpec((8, BLK), lambda i: (0, i))
    sspec = pl.BlockSpec((1, 1, BLK), lambda i: (i, 0, 0))
    seg_shape = jax.ShapeDtypeStruct((NBLK, 1, BLK), jnp.int32)
    a_shape = jax.ShapeDtypeStruct((8, NPTOT), jnp.float32)
    return pl.pallas_call(
        _prep_body,
        grid=(NBLK,),
        in_specs=[pspec],
        out_specs=[sspec, aspec, sspec, aspec],
        out_shape=[seg_shape, a_shape, seg_shape, a_shape],
    )(P)


def _zero_acc(acc):
    zeros16 = jnp.zeros((16,), jnp.float32)

    @pl.loop(0, NSEG // 128)
    def _z(i):
        for u in range(8):
            acc[pl.ds(i * 128 + u * 16, 16)] = zeros16


def _scatter_grp(acc, seg_v, val_v):
    @plsc.parallel_loop(0, GRP, 16, unroll=16)
    def _i(off):
        plsc.addupdate_scatter(acc, [seg_v[pl.ds(off, 16)]],
                               val_v[pl.ds(off, 16)])


def _gather_grp(acc, seg_v, out_v):
    @plsc.parallel_loop(0, GRP, 16, unroll=8)
    def _i(off):
        out_v[pl.ds(off, 16)] = plsc.load_gather(
            acc, [seg_v[pl.ds(off, 16)]])


def _grp_ring(b, srcs, bufs, sems, body):
    def start(g, k):
        for src, bb, ss in zip(srcs, bufs, sems):
            pltpu.async_copy(src(g), bb[k], ss[k])

    def wait(g, k):
        for src, bb, ss in zip(srcs, bufs, sems):
            pltpu.make_async_copy(src(g), bb[k], ss[k]).wait()

    start(0, 0)

    @pl.loop(0, NGRP // 2)
    def _gp(gp):
        g = gp * 2
        start(g + 1, 1)
        wait(g, 0)
        body(0, g)

        @pl.when(gp + 1 < NGRP // 2)
        def _nxt():
            start(g + 2, 0)

        wait(g + 1, 1)
        body(1, g + 1)


@functools.cache
def _make_stats():
    @functools.partial(
        pl.kernel,
        mesh=_mesh(),
        compiler_params=_SC_PARAMS,
        out_type=(jax.ShapeDtypeStruct((NBATCH, 4, NSEG), jnp.float32),
                  jax.ShapeDtypeStruct((8, NPTOT), jnp.float32)) * 2,
        scratch_types=[
            pltpu.VMEM((GRP,), jnp.int32),
            pltpu.VMEM((GRP,), jnp.int32),
            pltpu.VMEM((GRP,), jnp.float32),
            pltpu.VMEM((GRP,), jnp.float32),
            pltpu.VMEM((GRP,), jnp.float32),
            pltpu.VMEM((GRP,), jnp.float32),
            pltpu.VMEM((NSEG,), jnp.float32),
            pltpu.SemaphoreType.DMA,
            pltpu.SemaphoreType.DMA,
            pltpu.SemaphoreType.DMA,
            pltpu.SemaphoreType.DMA,
            pltpu.SemaphoreType.DMA,
            pltpu.SemaphoreType.DMA,
        ],
    )
    def stats_k(a0, a1, s0, s1, S0, G0, S1, G1,
                seg_a, seg_b, val_a, val_b, out_a, out_b, acc,
                sem_sa, sem_sb, sem_va, sem_vb, sem_oa, sem_ob):
        cid = lax.axis_index("c")
        tid = lax.axis_index("s")
        b = tid // 4
        ch = tid % 4

        def run(a_hbm, s_hbm, S_hbm, G_hbm):
            _zero_acc(acc)
            seg_src = lambda g: s_hbm.at[b, pl.ds(g * GRP, GRP)]
            val_src = lambda g: a_hbm.at[ch, pl.ds(b * NPB + g * GRP, GRP)]

            def sc_body(k, g):
                _scatter_grp(acc, (seg_a, seg_b)[k], (val_a, val_b)[k])

            _grp_ring(b, [seg_src, val_src],
                      [(seg_a, seg_b), (val_a, val_b)],
                      [(sem_sa, sem_sb), (sem_va, sem_vb)], sc_body)
            pltpu.sync_copy(acc, S_hbm.at[b, ch])

            def g_dst(g):
                return G_hbm.at[ch, pl.ds(b * NPB + g * GRP, GRP)]

            def ga_body(k, g):
                ov = (out_a, out_b)[k]
                so = (sem_oa, sem_ob)[k]
                @pl.when(g >= 2)
                def _d():
                    pltpu.make_async_copy(ov, g_dst(g - 2), so).wait()
                _gather_grp(acc, (seg_a, seg_b)[k], ov)
                pltpu.async_copy(ov, g_dst(g), so)

            _grp_ring(b, [seg_src],
                      [(seg_a, seg_b)],
                      [(sem_sa, sem_sb)], ga_body)
            pltpu.make_async_copy(out_a, g_dst(NGRP - 2), sem_oa).wait()
            pltpu.make_async_copy(out_b, g_dst(NGRP - 1), sem_ob).wait()

        @pl.when(jnp.logical_and(cid == 0, tid < 8))
        def _c0():
            run(a0, s0, S0, G0)

        @pl.when(jnp.logical_and(cid == 1, tid < 8))
        def _c1():
            run(a1, s1, S1, G1)

    return stats_k


def _pfn_body(a0, g0, a1, g1, wt, f0_ref, f1_ref):
    w = wt[...]

    def one(a, g, f_ref):
        ones = a[0:1, :]
        x = a[1:2, :]
        y = a[2:3, :]
        z = a[3:4, :]
        cxc = a[4:5, :]
        cyc = a[5:6, :]
        cnt = jnp.maximum(g[0:1, :], 1.0)
        mx = g[1:2, :] / cnt
        my = g[2:3, :] / cnt
        mz = g[3:4, :] / cnt
        zpad = jnp.zeros((7, BLK), jnp.float32)
        feats = jnp.concatenate(
            [x, y, z, x - mx, y - my, z - mz, x - cxc, y - cyc, zpad, ones],
            axis=0)
        yv = jnp.dot(w, feats, preferred_element_type=jnp.float32)
        f_ref[...] = jnp.where(ones > 0.0, jnp.maximum(yv, 0.0), 0.0)

    one(a0[...], g0[...], f0_ref)
    one(a1[...], g1[...], f1_ref)


def _pfn(A0, G0, A1, G1, WT):
    pspec = pl.BlockSpec((8, BLK), lambda i: (0, i))
    wspec = pl.BlockSpec((C, 16), lambda i: (0, 0))
    fspec = pl.BlockSpec((C, BLK), lambda i: (0, i))
    f_shape = jax.ShapeDtypeStruct((C, NPTOT), jnp.float32)
    return pl.pallas_call(
        _pfn_body,
        grid=(NBLK,),
        in_specs=[pspec, pspec, pspec, pspec, wspec],
        out_specs=[fspec, fspec],
        out_shape=[f_shape, f_shape],
    )(A0, G0, A1, G1, WT)


@functools.cache
def _make_fsum():
    @functools.partial(
        pl.kernel,
        mesh=_mesh(),
        compiler_params=_SC_PARAMS,
        out_type=(jax.ShapeDtypeStruct((NBATCH, C, NSEG), jnp.float32),) * 2,
        scratch_types=[
            pltpu.VMEM((GRP,), jnp.int32),
            pltpu.VMEM((GRP,), jnp.int32),
            pltpu.VMEM((GRP,), jnp.float32),
            pltpu.VMEM((GRP,), jnp.float32),
            pltpu.VMEM((NSEG,), jnp.float32),
            pltpu.SemaphoreType.DMA,
            pltpu.SemaphoreType.DMA,
            pltpu.SemaphoreType.DMA,
            pltpu.SemaphoreType.DMA,
        ],
    )
    def fsum_k(f0, f1, s0, s1, o0, o1,
               seg_a, seg_b, val_a, val_b, acc,
               sem_sa, sem_sb, sem_va, sem_vb):
        cid = lax.axis_index("c")
        tid = lax.axis_index("s")
        b = tid // 8
        chbase = tid % 8

        def run(f_hbm, s_hbm, o_hbm):
            @pl.loop(0, C // 8)
            def _r(r):
                ch = r * 8 + chbase
                _zero_acc(acc)
                seg_src = lambda g: s_hbm.at[b, pl.ds(g * GRP, GRP)]
                val_src = lambda g: f_hbm.at[ch, pl.ds(b * NPB + g * GRP,
                                                       GRP)]

                def sc_body(k, g):
                    _scatter_grp(acc, (seg_a, seg_b)[k], (val_a, val_b)[k])

                _grp_ring(b, [seg_src, val_src],
                          [(seg_a, seg_b), (val_a, val_b)],
                          [(sem_sa, sem_sb), (sem_va, sem_vb)], sc_body)
                pltpu.sync_copy(acc, o_hbm.at[b, ch])

        @pl.when(cid == 0)
        def _c0():
            run(f0, s0, o0)

        @pl.when(cid == 1)
        def _c1():
            run(f1, s1, o1)

    return fsum_k


def _combine_body(sum0, sum1, s0, s1, out_ref):
    cnt0 = jnp.maximum(s0[0, 0:1, :], 1.0)
    cnt1 = jnp.maximum(s1[0, 0:1, :], 1.0)
    y = sum1[0] / cnt1 - sum0[0] / cnt0
    out_ref[...] = y.T


def _combine(sums0, sums1, S0, S1):
    sumspec = pl.BlockSpec((1, C, SB), lambda b, j: (b, 0, j))
    sspec = pl.BlockSpec((1, 4, SB), lambda b, j: (b, 0, j))
    ospec = pl.BlockSpec((SB, C), lambda b, j: (b * (NSEG // SB) + j, 0))
    return pl.pallas_call(
        _combine_body,
        grid=(NBATCH, NSEG // SB),
        in_specs=[sumspec, sumspec, sspec, sspec],
        out_specs=ospec,
        out_shape=jax.ShapeDtypeStruct((NBATCH * NSEG, C), jnp.float32),
    )(sums0, sums1, S0, S1)


def kernel(pc0s, pc1s, W, b):
    pcs = jnp.stack([pc0s, pc1s])
    pcp = jnp.pad(pcs, ((0, 0), (0, 0), (0, NPB - N), (0, 0)))
    P = jnp.transpose(pcp, (0, 3, 1, 2)).reshape(2, 3, NPTOT)
    P = jnp.pad(P, ((0, 0), (0, 5), (0, 0)))
    WT = jnp.concatenate(
        [W, jnp.zeros((7, C), jnp.float32), b[None, :]], axis=0).T

    sg0, A0, sg1, A1 = _prep(P)
    seg0 = sg0.reshape(NBATCH, NPB)
    seg1 = sg1.reshape(NBATCH, NPB)

    S0, G0, S1, G1 = _make_stats()(A0, A1, seg0, seg1)
    F0, F1 = _pfn(A0, G0, A1, G1, WT)
    sums0, sums1 = _make_fsum()(F0, F1, seg0, seg1)
    return _combine(sums0, sums1, S0, S1)


# speedup vs baseline: 1.0041x; 1.0041x over previous
import functools

import jax
import jax.numpy as jnp
from jax import lax
from jax.experimental import pallas as pl
from jax.experimental.pallas import tpu as pltpu
from jax.experimental.pallas import tpu_sc as plsc

VX, VY = 0.4, 0.4
XMIN, YMIN = -51.2, -51.2
GX, GY = 256, 256
C = 64
NBATCH = 2
N = 100000
NSEG = GX * GY

NS = 16
NGRP = 16
GRP = 6272
NPB = NGRP * GRP
NPTOT = NBATCH * NPB
BLK = 2048
NBLK = NPTOT // BLK
BLKS_PER_BATCH = NPB // BLK
SB = 512

_SC_PARAMS = pltpu.CompilerParams(
    use_tc_tiling_on_sc=False, needs_layout_passes=False)


@functools.cache
def _mesh():
    return plsc.VectorSubcoreMesh(
        core_axis_name="c", subcore_axis_name="s",
        num_cores=2, num_subcores=NS)


def _prep_body(p0, s0_ref, a0_ref, s1_ref, a1_ref):
    i = pl.program_id(0)
    inb = (i % BLKS_PER_BATCH) * BLK
    lane = lax.broadcasted_iota(jnp.int32, (1, BLK), 1)
    valid = (inb + lane) < N

    def one(p, s_ref, a_ref):
        x = p[0:1, :]
        y = p[1:2, :]
        z = p[2:3, :]
        cx = jnp.clip(jnp.floor((x - XMIN) / VX).astype(jnp.int32), 0, GX - 1)
        cy = jnp.clip(jnp.floor((y - YMIN) / VY).astype(jnp.int32), 0, GY - 1)
        s_ref[0] = jnp.where(valid, cy * GX + cx, 0)
        cxc = (cx.astype(jnp.float32) + 0.5) * VX + XMIN
        cyc = (cy.astype(jnp.float32) + 0.5) * VY + YMIN
        ones = jnp.ones((1, BLK), jnp.float32)
        zpad = jnp.zeros((2, BLK), jnp.float32)
        a = jnp.concatenate([ones, x, y, z, cxc, cyc, zpad], axis=0)
        a_ref[...] = jnp.where(valid, a, 0.0)

    one(p0[0], s0_ref, a0_ref)
    one(p0[1], s1_ref, a1_ref)


def _prep(P):
    pspec = pl.BlockSpec((2, 8, BLK), lambda i: (0, 0, i))
    aspec = pl.BlockSpec((8, BLK), lambda i: (0, i))
    sspec = pl.BlockSpec((1, 1, BLK), lambda i: (i, 0, 0))
    seg_shape = jax.ShapeDtypeStruct((NBLK, 1, BLK), jnp.int32)
    a_shape = jax.ShapeDtypeStruct((8, NPTOT), jnp.float32)
    return pl.pallas_call(
        _prep_body,
        grid=(NBLK,),
        in_specs=[pspec],
        out_specs=[sspec, aspec, sspec, aspec],
        out_shape=[seg_shape, a_shape, seg_shape, a_shape],
    )(P)


def _zero_acc(acc):
    zeros16 = jnp.zeros((16,), jnp.float32)

    @pl.loop(0, NSEG // 128)
    def _z(i):
        for u in range(8):
            acc[pl.ds(i * 128 + u * 16, 16)] = zeros16


def _scatter_grp(acc, seg_v, val_v):
    @plsc.parallel_loop(0, GRP, 16, unroll=8)
    def _i(off):
        plsc.addupdate_scatter(acc, [seg_v[pl.ds(off, 16)]],
                               val_v[pl.ds(off, 16)])


def _gather_grp(acc, seg_v, out_v):
    @plsc.parallel_loop(0, GRP, 16, unroll=8)
    def _i(off):
        out_v[pl.ds(off, 16)] = plsc.load_gather(
            acc, [seg_v[pl.ds(off, 16)]])


def _grp_ring(b, srcs, bufs, sems, body):
    def start(g, k):
        for src, bb, ss in zip(srcs, bufs, sems):
            pltpu.async_copy(src(g), bb[k], ss[k])

    def wait(g, k):
        for src, bb, ss in zip(srcs, bufs, sems):
            pltpu.make_async_copy(src(g), bb[k], ss[k]).wait()

    start(0, 0)

    @pl.loop(0, NGRP // 2)
    def _gp(gp):
        g = gp * 2
        start(g + 1, 1)
        wait(g, 0)
        body(0, g)

        @pl.when(gp + 1 < NGRP // 2)
        def _nxt():
            start(g + 2, 0)

        wait(g + 1, 1)
        body(1, g + 1)


@functools.cache
def _make_stats():
    @functools.partial(
        pl.kernel,
        mesh=_mesh(),
        compiler_params=_SC_PARAMS,
        out_type=(jax.ShapeDtypeStruct((NBATCH, 4, NSEG), jnp.float32),
                  jax.ShapeDtypeStruct((8, NPTOT), jnp.float32)) * 2,
        scratch_types=[
            pltpu.VMEM((GRP,), jnp.int32),
            pltpu.VMEM((GRP,), jnp.int32),
            pltpu.VMEM((GRP,), jnp.float32),
            pltpu.VMEM((GRP,), jnp.float32),
            pltpu.VMEM((GRP,), jnp.float32),
            pltpu.VMEM((GRP,), jnp.float32),
            pltpu.VMEM((NSEG,), jnp.float32),
            pltpu.SemaphoreType.DMA,
            pltpu.SemaphoreType.DMA,
            pltpu.SemaphoreType.DMA,
            pltpu.SemaphoreType.DMA,
            pltpu.SemaphoreType.DMA,
            pltpu.SemaphoreType.DMA,
        ],
    )
    def stats_k(a0, a1, s0, s1, S0, G0, S1, G1,
                seg_a, seg_b, val_a, val_b, out_a, out_b, acc,
                sem_sa, sem_sb, sem_va, sem_vb, sem_oa, sem_ob):
        cid = lax.axis_index("c")
        tid = lax.axis_index("s")
        b = tid // 4
        ch = tid % 4

        def run(a_hbm, s_hbm, S_hbm, G_hbm):
            _zero_acc(acc)
            seg_src = lambda g: s_hbm.at[b, pl.ds(g * GRP, GRP)]
            val_src = lambda g: a_hbm.at[ch, pl.ds(b * NPB + g * GRP, GRP)]

            def sc_body(k, g):
                _scatter_grp(acc, (seg_a, seg_b)[k], (val_a, val_b)[k])

            _grp_ring(b, [seg_src, val_src],
                      [(seg_a, seg_b), (val_a, val_b)],
                      [(sem_sa, sem_sb), (sem_va, sem_vb)], sc_body)
            pltpu.sync_copy(acc, S_hbm.at[b, ch])

            def g_dst(g):
                return G_hbm.at[ch, pl.ds(b * NPB + g * GRP, GRP)]

            def ga_body(k, g):
                ov = (out_a, out_b)[k]
                so = (sem_oa, sem_ob)[k]
                @pl.when(g >= 2)
                def _d():
                    pltpu.make_async_copy(ov, g_dst(g - 2), so).wait()
                _gather_grp(acc, (seg_a, seg_b)[k], ov)
                pltpu.async_copy(ov, g_dst(g), so)

            _grp_ring(b, [seg_src],
                      [(seg_a, seg_b)],
                      [(sem_sa, sem_sb)], ga_body)
            pltpu.make_async_copy(out_a, g_dst(NGRP - 2), sem_oa).wait()
            pltpu.make_async_copy(out_b, g_dst(NGRP - 1), sem_ob).wait()

        @pl.when(jnp.logical_and(cid == 0, tid < 8))
        def _c0():
            run(a0, s0, S0, G0)

        @pl.when(jnp.logical_and(cid == 1, tid < 8))
        def _c1():
            run(a1, s1, S1, G1)

    return stats_k


def _pfn_body(a0, g0, a1, g1, wt, f0_ref, f1_ref):
    w = wt[...]

    def one(a, g, f_ref):
        ones = a[0:1, :]
        x = a[1:2, :]
        y = a[2:3, :]
        z = a[3:4, :]
        cxc = a[4:5, :]
        cyc = a[5:6, :]
        cnt = jnp.maximum(g[0:1, :], 1.0)
        mx = g[1:2, :] / cnt
        my = g[2:3, :] / cnt
        mz = g[3:4, :] / cnt
        zpad = jnp.zeros((7, BLK), jnp.float32)
        feats = jnp.concatenate(
            [x, y, z, x - mx, y - my, z - mz, x - cxc, y - cyc, zpad, ones],
            axis=0)
        yv = jnp.dot(w, feats, preferred_element_type=jnp.float32)
        f_ref[...] = jnp.where(ones > 0.0, jnp.maximum(yv, 0.0), 0.0)

    one(a0[...], g0[...], f0_ref)
    one(a1[...], g1[...], f1_ref)


def _pfn(A0, G0, A1, G1, WT):
    pspec = pl.BlockSpec((8, BLK), lambda i: (0, i))
    wspec = pl.BlockSpec((C, 16), lambda i: (0, 0))
    fspec = pl.BlockSpec((C, BLK), lambda i: (0, i))
    f_shape = jax.ShapeDtypeStruct((C, NPTOT), jnp.float32)
    return pl.pallas_call(
        _pfn_body,
        grid=(NBLK,),
        in_specs=[pspec, pspec, pspec, pspec, wspec],
        out_specs=[fspec, fspec],
        out_shape=[f_shape, f_shape],
    )(A0, G0, A1, G1, WT)


@functools.cache
def _make_fsum():
    @functools.partial(
        pl.kernel,
        mesh=_mesh(),
        compiler_params=_SC_PARAMS,
        out_type=(jax.ShapeDtypeStruct((NBATCH, C, NSEG), jnp.float32),) * 2,
        scratch_types=[
            pltpu.VMEM((GRP,), jnp.int32),
            pltpu.VMEM((GRP,), jnp.int32),
            pltpu.VMEM((GRP,), jnp.float32),
            pltpu.VMEM((GRP,), jnp.float32),
            pltpu.VMEM((NSEG,), jnp.float32),
            pltpu.SemaphoreType.DMA,
            pltpu.SemaphoreType.DMA,
            pltpu.SemaphoreType.DMA,
            pltpu.SemaphoreType.DMA,
        ],
    )
    def fsum_k(f0, f1, s0, s1, o0, o1,
               seg_a, seg_b, val_a, val_b, acc,
               sem_sa, sem_sb, sem_va, sem_vb):
        cid = lax.axis_index("c")
        tid = lax.axis_index("s")
        b = tid // 8
        chbase = tid % 8

        def run(f_hbm, s_hbm, o_hbm):
            @pl.loop(0, C // 8)
            def _r(r):
                ch = r * 8 + chbase
                _zero_acc(acc)
                seg_src = lambda g: s_hbm.at[b, pl.ds(g * GRP, GRP)]
                val_src = lambda g: f_hbm.at[ch, pl.ds(b * NPB + g * GRP,
                                                       GRP)]

                def sc_body(k, g):
                    _scatter_grp(acc, (seg_a, seg_b)[k], (val_a, val_b)[k])

                _grp_ring(b, [seg_src, val_src],
                          [(seg_a, seg_b), (val_a, val_b)],
                          [(sem_sa, sem_sb), (sem_va, sem_vb)], sc_body)
                pltpu.sync_copy(acc, o_hbm.at[b, ch])

        @pl.when(cid == 0)
        def _c0():
            run(f0, s0, o0)

        @pl.when(cid == 1)
        def _c1():
            run(f1, s1, o1)

    return fsum_k


def _combine_body(sum0, sum1, s0, s1, out_ref):
    cnt0 = jnp.maximum(s0[0, 0:1, :], 1.0)
    cnt1 = jnp.maximum(s1[0, 0:1, :], 1.0)
    y = sum1[0] / cnt1 - sum0[0] / cnt0
    out_ref[...] = y.T


def _combine(sums0, sums1, S0, S1):
    sumspec = pl.BlockSpec((1, C, SB), lambda b, j: (b, 0, j))
    sspec = pl.BlockSpec((1, 4, SB), lambda b, j: (b, 0, j))
    ospec = pl.BlockSpec((SB, C), lambda b, j: (b * (NSEG // SB) + j, 0))
    return pl.pallas_call(
        _combine_body,
        grid=(NBATCH, NSEG // SB),
        in_specs=[sumspec, sumspec, sspec, sspec],
        out_specs=ospec,
        out_shape=jax.ShapeDtypeStruct((NBATCH * NSEG, C), jnp.float32),
    )(sums0, sums1, S0, S1)


def kernel(pc0s, pc1s, W, b):
    pcs = jnp.stack([pc0s, pc1s])
    pcp = jnp.pad(pcs, ((0, 0), (0, 0), (0, NPB - N), (0, 0)))
    P = jnp.transpose(pcp, (0, 3, 1, 2)).reshape(2, 3, NPTOT)
    P = jnp.pad(P, ((0, 0), (0, 5), (0, 0)))
    WT = jnp.concatenate(
        [W, jnp.zeros((7, C), jnp.float32), b[None, :]], axis=0).T

    sg0, A0, sg1, A1 = _prep(P)
    seg0 = sg0.reshape(NBATCH, NPB)
    seg1 = sg1.reshape(NBATCH, NPB)

    S0, G0, S1, G1 = _make_stats()(A0, A1, seg0, seg1)
    F0, F1 = _pfn(A0, G0, A1, G1, WT)
    sums0, sums1 = _make_fsum()(F0, F1, seg0, seg1)
    return _combine(sums0, sums1, S0, S1)


# speedup vs baseline: 1.0165x; 1.0124x over previous
import functools

import jax
import jax.numpy as jnp
from jax import lax
from jax.experimental import pallas as pl
from jax.experimental.pallas import tpu as pltpu
from jax.experimental.pallas import tpu_sc as plsc

VX, VY = 0.4, 0.4
XMIN, YMIN = -51.2, -51.2
GX, GY = 256, 256
C = 64
NBATCH = 2
N = 100000
NSEG = GX * GY

NS = 16
NGRP = 16
GRP = 6272
NPB = NGRP * GRP
NPTOT = NBATCH * NPB
BLK = 2048
NBLK = NPTOT // BLK
BLKS_PER_BATCH = NPB // BLK
SB = 512

_SC_PARAMS = pltpu.CompilerParams(
    use_tc_tiling_on_sc=False, needs_layout_passes=False)


@functools.cache
def _mesh():
    return plsc.VectorSubcoreMesh(
        core_axis_name="c", subcore_axis_name="s",
        num_cores=2, num_subcores=NS)


def _prep_body(p0, p1, s0_ref, a0_ref, s1_ref, a1_ref):
    i = pl.program_id(0)
    inb = (i % BLKS_PER_BATCH) * BLK
    lane = lax.broadcasted_iota(jnp.int32, (1, BLK), 1)
    valid = (inb + lane) < N

    def one(p, s_ref, a_ref):
        x = p[0:1, :]
        y = p[1:2, :]
        z = p[2:3, :]
        cx = jnp.clip(jnp.floor((x - XMIN) / VX).astype(jnp.int32), 0, GX - 1)
        cy = jnp.clip(jnp.floor((y - YMIN) / VY).astype(jnp.int32), 0, GY - 1)
        s_ref[0] = jnp.where(valid, cy * GX + cx, 0)
        cxc = (cx.astype(jnp.float32) + 0.5) * VX + XMIN
        cyc = (cy.astype(jnp.float32) + 0.5) * VY + YMIN
        ones = jnp.ones((1, BLK), jnp.float32)
        zpad = jnp.zeros((2, BLK), jnp.float32)
        a = jnp.concatenate([ones, x, y, z, cxc, cyc, zpad], axis=0)
        a_ref[...] = jnp.where(valid, a, 0.0)

    one(p0[...], s0_ref, a0_ref)
    one(p1[...], s1_ref, a1_ref)


def _prep(P0, P1):
    pspec = pl.BlockSpec((8, BLK), lambda i: (0, i))
    sspec = pl.BlockSpec((1, 1, BLK), lambda i: (i, 0, 0))
    seg_shape = jax.ShapeDtypeStruct((NBLK, 1, BLK), jnp.int32)
    a_shape = jax.ShapeDtypeStruct((8, NPTOT), jnp.float32)
    return pl.pallas_call(
        _prep_body,
        grid=(NBLK,),
        in_specs=[pspec, pspec],
        out_specs=[sspec, pspec, sspec, pspec],
        out_shape=[seg_shape, a_shape, seg_shape, a_shape],
    )(P0, P1)


def _zero_acc(acc):
    zeros16 = jnp.zeros((16,), jnp.float32)

    @pl.loop(0, NSEG // 128)
    def _z(i):
        for u in range(8):
            acc[pl.ds(i * 128 + u * 16, 16)] = zeros16


def _scatter_grp(acc, seg_v, val_v):
    @plsc.parallel_loop(0, GRP, 16, unroll=8)
    def _i(off):
        plsc.addupdate_scatter(acc, [seg_v[pl.ds(off, 16)]],
                               val_v[pl.ds(off, 16)])


def _gather_grp(acc, seg_v, out_v):
    @plsc.parallel_loop(0, GRP, 16, unroll=8)
    def _i(off):
        out_v[pl.ds(off, 16)] = plsc.load_gather(
            acc, [seg_v[pl.ds(off, 16)]])


def _grp_ring(b, srcs, bufs, sems, body):
    def start(g, k):
        for src, bb, ss in zip(srcs, bufs, sems):
            pltpu.async_copy(src(g), bb[k], ss[k])

    def wait(g, k):
        for src, bb, ss in zip(srcs, bufs, sems):
            pltpu.make_async_copy(src(g), bb[k], ss[k]).wait()

    start(0, 0)

    @pl.loop(0, NGRP // 2)
    def _gp(gp):
        g = gp * 2
        start(g + 1, 1)
        wait(g, 0)
        body(0, g)

        @pl.when(gp + 1 < NGRP // 2)
        def _nxt():
            start(g + 2, 0)

        wait(g + 1, 1)
        body(1, g + 1)


@functools.cache
def _make_stats():
    @functools.partial(
        pl.kernel,
        mesh=_mesh(),
        compiler_params=_SC_PARAMS,
        out_type=(jax.ShapeDtypeStruct((NBATCH, 4, NSEG), jnp.float32),
                  jax.ShapeDtypeStruct((8, NPTOT), jnp.float32)) * 2,
        scratch_types=[
            pltpu.VMEM((GRP,), jnp.int32),
            pltpu.VMEM((GRP,), jnp.int32),
            pltpu.VMEM((GRP,), jnp.float32),
            pltpu.VMEM((GRP,), jnp.float32),
            pltpu.VMEM((GRP,), jnp.float32),
            pltpu.VMEM((GRP,), jnp.float32),
            pltpu.VMEM((NSEG,), jnp.float32),
            pltpu.SemaphoreType.DMA,
            pltpu.SemaphoreType.DMA,
            pltpu.SemaphoreType.DMA,
            pltpu.SemaphoreType.DMA,
            pltpu.SemaphoreType.DMA,
            pltpu.SemaphoreType.DMA,
        ],
    )
    def stats_k(a0, a1, s0, s1, S0, G0, S1, G1,
                seg_a, seg_b, val_a, val_b, out_a, out_b, acc,
                sem_sa, sem_sb, sem_va, sem_vb, sem_oa, sem_ob):
        cid = lax.axis_index("c")
        tid = lax.axis_index("s")
        b = tid // 4
        ch = tid % 4

        def run(a_hbm, s_hbm, S_hbm, G_hbm):
            _zero_acc(acc)
            seg_src = lambda g: s_hbm.at[b, pl.ds(g * GRP, GRP)]
            val_src = lambda g: a_hbm.at[ch, pl.ds(b * NPB + g * GRP, GRP)]

            def sc_body(k, g):
                _scatter_grp(acc, (seg_a, seg_b)[k], (val_a, val_b)[k])

            _grp_ring(b, [seg_src, val_src],
                      [(seg_a, seg_b), (val_a, val_b)],
                      [(sem_sa, sem_sb), (sem_va, sem_vb)], sc_body)
            pltpu.sync_copy(acc, S_hbm.at[b, ch])

            def g_dst(g):
                return G_hbm.at[ch, pl.ds(b * NPB + g * GRP, GRP)]

            def ga_body(k, g):
                ov = (out_a, out_b)[k]
                so = (sem_oa, sem_ob)[k]
                @pl.when(g >= 2)
                def _d():
                    pltpu.make_async_copy(ov, g_dst(g - 2), so).wait()
                _gather_grp(acc, (seg_a, seg_b)[k], ov)
                pltpu.async_copy(ov, g_dst(g), so)

            _grp_ring(b, [seg_src],
                      [(seg_a, seg_b)],
                      [(sem_sa, sem_sb)], ga_body)
            pltpu.make_async_copy(out_a, g_dst(NGRP - 2), sem_oa).wait()
            pltpu.make_async_copy(out_b, g_dst(NGRP - 1), sem_ob).wait()

        @pl.when(jnp.logical_and(cid == 0, tid < 8))
        def _c0():
            run(a0, s0, S0, G0)

        @pl.when(jnp.logical_and(cid == 1, tid < 8))
        def _c1():
            run(a1, s1, S1, G1)

    return stats_k


def _pfn_body(a0, g0, a1, g1, wt, f0_ref, f1_ref):
    w = wt[...]

    def one(a, g, f_ref):
        ones = a[0:1, :]
        x = a[1:2, :]
        y = a[2:3, :]
        z = a[3:4, :]
        cxc = a[4:5, :]
        cyc = a[5:6, :]
        cnt = jnp.maximum(g[0:1, :], 1.0)
        mx = g[1:2, :] / cnt
        my = g[2:3, :] / cnt
        mz = g[3:4, :] / cnt
        zpad = jnp.zeros((7, BLK), jnp.float32)
        feats = jnp.concatenate(
            [x, y, z, x - mx, y - my, z - mz, x - cxc, y - cyc, zpad, ones],
            axis=0)
        yv = jnp.dot(w, feats, preferred_element_type=jnp.float32)
        f_ref[...] = jnp.where(ones > 0.0, jnp.maximum(yv, 0.0), 0.0)

    one(a0[...], g0[...], f0_ref)
    one(a1[...], g1[...], f1_ref)


def _pfn(A0, G0, A1, G1, WT):
    pspec = pl.BlockSpec((8, BLK), lambda i: (0, i))
    wspec = pl.BlockSpec((C, 16), lambda i: (0, 0))
    fspec = pl.BlockSpec((C, BLK), lambda i: (0, i))
    f_shape = jax.ShapeDtypeStruct((C, NPTOT), jnp.float32)
    return pl.pallas_call(
        _pfn_body,
        grid=(NBLK,),
        in_specs=[pspec, pspec, pspec, pspec, wspec],
        out_specs=[fspec, fspec],
        out_shape=[f_shape, f_shape],
    )(A0, G0, A1, G1, WT)


@functools.cache
def _make_fsum():
    @functools.partial(
        pl.kernel,
        mesh=_mesh(),
        compiler_params=_SC_PARAMS,
        out_type=(jax.ShapeDtypeStruct((NBATCH, C, NSEG), jnp.float32),) * 2,
        scratch_types=[
            pltpu.VMEM((GRP,), jnp.int32),
            pltpu.VMEM((GRP,), jnp.int32),
            pltpu.VMEM((GRP,), jnp.float32),
            pltpu.VMEM((GRP,), jnp.float32),
            pltpu.VMEM((NSEG,), jnp.float32),
            pltpu.SemaphoreType.DMA,
            pltpu.SemaphoreType.DMA,
            pltpu.SemaphoreType.DMA,
            pltpu.SemaphoreType.DMA,
        ],
    )
    def fsum_k(f0, f1, s0, s1, o0, o1,
               seg_a, seg_b, val_a, val_b, acc,
               sem_sa, sem_sb, sem_va, sem_vb):
        cid = lax.axis_index("c")
        tid = lax.axis_index("s")
        b = tid // 8
        chbase = tid % 8

        def run(f_hbm, s_hbm, o_hbm):
            @pl.loop(0, C // 8)
            def _r(r):
                ch = r * 8 + chbase
                _zero_acc(acc)
                seg_src = lambda g: s_hbm.at[b, pl.ds(g * GRP, GRP)]
                val_src = lambda g: f_hbm.at[ch, pl.ds(b * NPB + g * GRP,
                                                       GRP)]

                def sc_body(k, g):
                    _scatter_grp(acc, (seg_a, seg_b)[k], (val_a, val_b)[k])

                _grp_ring(b, [seg_src, val_src],
                          [(seg_a, seg_b), (val_a, val_b)],
                          [(sem_sa, sem_sb), (sem_va, sem_vb)], sc_body)
                pltpu.sync_copy(acc, o_hbm.at[b, ch])

        @pl.when(cid == 0)
        def _c0():
            run(f0, s0, o0)

        @pl.when(cid == 1)
        def _c1():
            run(f1, s1, o1)

    return fsum_k


def _combine_body(sum0, sum1, s0, s1, out_ref):
    cnt0 = jnp.maximum(s0[0, 0:1, :], 1.0)
    cnt1 = jnp.maximum(s1[0, 0:1, :], 1.0)
    y = sum1[0] / cnt1 - sum0[0] / cnt0
    out_ref[...] = y.T


def _combine(sums0, sums1, S0, S1):
    sumspec = pl.BlockSpec((1, C, SB), lambda b, j: (b, 0, j))
    sspec = pl.BlockSpec((1, 4, SB), lambda b, j: (b, 0, j))
    ospec = pl.BlockSpec((SB, C), lambda b, j: (b * (NSEG // SB) + j, 0))
    return pl.pallas_call(
        _combine_body,
        grid=(NBATCH, NSEG // SB),
        in_specs=[sumspec, sumspec, sspec, sspec],
        out_specs=ospec,
        out_shape=jax.ShapeDtypeStruct((NBATCH * NSEG, C), jnp.float32),
    )(sums0, sums1, S0, S1)


def kernel(pc0s, pc1s, W, b):
    def planar(pc):
        pcp = jnp.pad(pc, ((0, 0), (0, NPB - N), (0, 0)))
        two = jnp.concatenate([pcp[0].T, pcp[1].T], axis=1)
        return jnp.pad(two, ((0, 5), (0, 0)))

    P0 = planar(pc0s)
    P1 = planar(pc1s)
    WT = jnp.concatenate(
        [W, jnp.zeros((7, C), jnp.float32), b[None, :]], axis=0).T

    sg0, A0, sg1, A1 = _prep(P0, P1)
    seg0 = sg0.reshape(NBATCH, NPB)
    seg1 = sg1.reshape(NBATCH, NPB)

    S0, G0, S1, G1 = _make_stats()(A0, A1, seg0, seg1)
    F0, F1 = _pfn(A0, G0, A1, G1, WT)
    sums0, sums1 = _make_fsum()(F0, F1, seg0, seg1)
    return _combine(sums0, sums1, S0, S1)
